# baseline (speedup 1.0000x reference)
import functools

import jax
import jax.numpy as jnp
from jax import lax
from jax.experimental import pallas as pl
from jax.experimental.pallas import tpu as pltpu
from jax.experimental.pallas import tpu_sc as plsc

_NSUB = 16
_NSC = 2
_LANES = 16


def _pre_body(x_ref, w_ref, b_ref, hb_ref, hr_ref):
    h = jnp.dot(x_ref[...], w_ref[...], preferred_element_type=jnp.float32)
    b = b_ref[...]
    hb_ref[...] = h + 0.5 * b
    hr_ref[...] = jnp.maximum(h + b, 0.0)


def _pre(x, W, b, block):
    M, D = x.shape
    assert M % block == 0
    out = jax.ShapeDtypeStruct((M, D), jnp.float32)
    return pl.pallas_call(
        _pre_body,
        grid=(M // block,),
        in_specs=[
            pl.BlockSpec((block, D), lambda i: (i, 0)),
            pl.BlockSpec((D, D), lambda i: (0, 0)),
            pl.BlockSpec((1, D), lambda i: (0, 0)),
        ],
        out_specs=[
            pl.BlockSpec((block, D), lambda i: (i, 0)),
            pl.BlockSpec((block, D), lambda i: (i, 0)),
        ],
        out_shape=[out, out],
    )(x, W, b.reshape(1, D))


def _mlp_block(x, w1_ref, b1_ref, g_ref, beta_ref, w2_ref, b2_ref):
    h = jnp.dot(x, w1_ref[...], preferred_element_type=jnp.float32) + b1_ref[...]
    mu = jnp.mean(h, axis=-1, keepdims=True)
    var = jnp.mean((h - mu) * (h - mu), axis=-1, keepdims=True)
    h = (h - mu) * lax.rsqrt(var + 1e-5) * g_ref[...] + beta_ref[...]
    h = jnp.maximum(h, 0.0)
    return jnp.dot(h, w2_ref[...], preferred_element_type=jnp.float32) + b2_ref[...]


def _out1_body(e1_ref, g01_ref, tri_ref, w111_ref, b111_ref,
               w1_ref, b1_ref, g_ref, beta_ref, w2_ref, b2_ref, o_ref):
    e1 = e1_ref[...]
    x = (e1 + jnp.maximum(g01_ref[...], 0.0)
         + jnp.dot(tri_ref[...], w111_ref[...], preferred_element_type=jnp.float32)
         + b111_ref[...])
    o_ref[...] = _mlp_block(x, w1_ref, b1_ref, g_ref, beta_ref, w2_ref, b2_ref) + e1


def _out1(e1, g01, tri_agg, W111, b111, W1, b1, g, beta, W2, b2, block):
    M, D = e1.shape
    assert M % block == 0
    row = lambda i: (i, 0)
    fixed = lambda i: (0, 0)
    vec = pl.BlockSpec((1, D), fixed)
    mat = pl.BlockSpec((D, D), fixed)
    return pl.pallas_call(
        _out1_body,
        grid=(M // block,),
        in_specs=[
            pl.BlockSpec((block, D), row),
            pl.BlockSpec((block, D), row),
            pl.BlockSpec((block, D), row),
            mat, vec, mat, vec, vec, vec, mat, vec,
        ],
        out_specs=pl.BlockSpec((block, D), row),
        out_shape=jax.ShapeDtypeStruct((M, D), jnp.float32),
    )(e1, g01, tri_agg, W111, b111.reshape(1, D), W1, b1.reshape(1, D),
      g.reshape(1, D), beta.reshape(1, D), W2, b2.reshape(1, D))


def _out0_body(e0_ref, p0_ref, p1_ref,
               w1_ref, b1_ref, g_ref, beta_ref, w2_ref, b2_ref, o_ref):
    e0 = e0_ref[...]
    x = e0 + p0_ref[...] + p1_ref[...]
    o_ref[...] = _mlp_block(x, w1_ref, b1_ref, g_ref, beta_ref, w2_ref, b2_ref) + e0


def _out0(e0, p0, p1, W1, b1, g, beta, W2, b2, block):
    M, D = e0.shape
    assert M % block == 0
    row = lambda i: (i, 0)
    fixed = lambda i: (0, 0)
    vec = pl.BlockSpec((1, D), fixed)
    mat = pl.BlockSpec((D, D), fixed)
    return pl.pallas_call(
        _out0_body,
        grid=(M // block,),
        in_specs=[
            pl.BlockSpec((block, D), row),
            pl.BlockSpec((block, D), row),
            pl.BlockSpec((block, D), row),
            mat, vec, vec, vec, mat, vec,
        ],
        out_specs=pl.BlockSpec((block, D), row),
        out_shape=jax.ShapeDtypeStruct((M, D), jnp.float32),
    )(e0, p0, p1, W1, b1.reshape(1, D), g.reshape(1, D), beta.reshape(1, D),
      W2, b2.reshape(1, D))


_ER = 80
_CW = 80


def _sc_edges(hb0, hr1, src, dst, zeros):
    N, D = hb0.shape
    E = src.shape[0]
    ntiles = _NSC * _NSUB
    epert = E // ntiles
    assert epert * ntiles == E and epert % _ER == 0
    nblk = epert // _ER
    assert N % _CW == 0
    nchunk = N // _CW
    jmax = -(-nchunk // _NSUB)

    mesh = plsc.VectorSubcoreMesh(core_axis_name="c", subcore_axis_name="s")

    @functools.partial(
        pl.kernel,
        out_type=[
            jax.ShapeDtypeStruct((E, D), jnp.float32),
            jax.ShapeDtypeStruct((_NSC * N, D), jnp.float32),
        ],
        mesh=mesh,
        scratch_types=[
            pltpu.VMEM((1, _ER), jnp.int32),
            pltpu.VMEM((1, _ER), jnp.int32),
            pltpu.VMEM((_ER, D), jnp.float32),
            pltpu.VMEM((_ER, D), jnp.float32),
            pltpu.VMEM((_ER, D), jnp.float32),
            pltpu.VMEM((_CW, D), jnp.float32),
            pltpu.VMEM_SHARED((N, D), jnp.float32),
        ],
    )
    def k(hb0_hbm, hr1_hbm, src_hbm, dst_hbm, z_hbm, g01_hbm, p_hbm,
          idx_s, idx_d, rows_s, rows_d, rows_h, bounce, acc):
        core = lax.axis_index("c")
        sub = lax.axis_index("s")
        wid = core * _NSUB + sub

        pltpu.sync_copy(z_hbm, bounce)
        for j in range(jmax):
            ch = sub + j * _NSUB

            @pl.when(ch < nchunk)
            def _():
                pltpu.sync_copy(bounce, acc.at[pl.ds(ch * _CW, _CW)])

        plsc.subcore_barrier()

        @pl.loop(0, nblk)
        def _blk(i):
            b0 = wid * epert + i * _ER
            pltpu.sync_copy(src_hbm.at[pl.ds(b0, _ER)], idx_s.at[0])
            pltpu.sync_copy(dst_hbm.at[pl.ds(b0, _ER)], idx_d.at[0])
            pltpu.sync_copy(hb0_hbm.at[idx_s.at[0]], rows_s)
            pltpu.sync_copy(hb0_hbm.at[idx_d.at[0]], rows_d)

            @pl.loop(0, _ER)
            def _r(r):
                for cc in range(D // _LANES):
                    sl = pl.ds(cc * _LANES, _LANES)
                    rows_s[r, sl] = rows_s[r, sl] + rows_d[r, sl]

            pltpu.sync_copy(rows_s, g01_hbm.at[pl.ds(b0, _ER)])
            pltpu.sync_copy(hr1_hbm.at[pl.ds(b0, _ER)], rows_h)
            pltpu.sync_copy(rows_h, acc.at[idx_s.at[0]], add=True)

        plsc.subcore_barrier()
        for j in range(jmax):
            ch = sub + j * _NSUB

            @pl.when(ch < nchunk)
            def _():
                r0 = ch * _CW
                pltpu.sync_copy(acc.at[pl.ds(r0, _CW)], bounce)
                pltpu.sync_copy(bounce, p_hbm.at[pl.ds(core * N + r0, _CW)])

    return k(hb0, hr1, src, dst, zeros)


_CH = 10000
_K = 80


def _sc_tri(hb1, t0, t1, t2, zeros):
    E, D = hb1.shape
    T = t0.shape[0]
    npass = E // (_NSC * _CH)
    assert npass * _NSC * _CH == E
    tpert = T // _NSUB
    assert tpert * _NSUB == T and tpert % _K == 0
    nblk = tpert // _K
    assert _CH % _CW == 0
    nchunk = _CH // _CW
    jmax = -(-nchunk // _NSUB)

    mesh = plsc.VectorSubcoreMesh(core_axis_name="c", subcore_axis_name="s")

    @functools.partial(
        pl.kernel,
        out_type=jax.ShapeDtypeStruct((E, D), jnp.float32),
        mesh=mesh,
        scratch_types=[
            pltpu.VMEM((1, _K), jnp.int32),
            pltpu.VMEM((1, _K), jnp.int32),
            pltpu.VMEM((1, _K), jnp.int32),
            pltpu.VMEM((_K, D), jnp.float32),
            pltpu.VMEM((_K, D), jnp.float32),
            pltpu.VMEM((_CW, D), jnp.float32),
            pltpu.VMEM_SHARED((_CH + 8, D), jnp.float32),
        ],
    )
    def k(hb_hbm, t0_hbm, t1_hbm, t2_hbm, z_hbm, out_hbm,
          bid, bia, bib, rows_a, rows_b, bounce, acc):
        core = lax.axis_index("c")
        sub = lax.axis_index("s")

        @pl.loop(0, npass)
        def _pass(p):
            lo = (_NSC * p + core) * _CH

            pltpu.sync_copy(z_hbm, bounce)
            for j in range(jmax):
                ch = sub + j * _NSUB

                @pl.when(ch < nchunk)
                def _():
                    pltpu.sync_copy(bounce, acc.at[pl.ds(ch * _CW, _CW)])
            plsc.subcore_barrier()

            @pl.loop(0, nblk)
            def _blk(i):
                tbase = sub * tpert + i * _K
                pltpu.sync_copy(t0_hbm.at[pl.ds(tbase, _K)], bid.at[0])
                pltpu.sync_copy(t1_hbm.at[pl.ds(tbase, _K)], bia.at[0])
                pltpu.sync_copy(t2_hbm.at[pl.ds(tbase, _K)], bib.at[0])
                pltpu.sync_copy(hb_hbm.at[bia.at[0]], rows_a)
                pltpu.sync_copy(hb_hbm.at[bib.at[0]], rows_b)

                @pl.loop(0, _K // _LANES)
                def _v(v):
                    sl = (pl.ds(0, 1), pl.ds(v * _LANES, _LANES))
                    local = bid[sl] - lo
                    m = (local >= 0) & (local < _CH)
                    bid[sl] = jnp.where(m, local, _CH)

                @pl.loop(0, _K)
                def _r(r):
                    for cc in range(D // _LANES):
                        sl = (pl.ds(r, 1), pl.ds(cc * _LANES, _LANES))
                        rows_a[sl] = jnp.maximum(rows_a[sl] + rows_b[sl], 0.0)

                pltpu.sync_copy(rows_a, acc.at[bid.at[0]], add=True)

            plsc.subcore_barrier()
            for j in range(jmax):
                ch = sub + j * _NSUB

                @pl.when(ch < nchunk)
                def _():
                    r0 = ch * _CW
                    pltpu.sync_copy(acc.at[pl.ds(r0, _CW)], bounce)
                    pltpu.sync_copy(bounce, out_hbm.at[pl.ds(lo + r0, _CW)])
            plsc.subcore_barrier()

    return k(hb1, t0, t1, t2, zeros)


def kernel(edge_attr0, edge_attr1, edge_index1, tri111, inverse_edge1,
           W_inner, b_inner, W_111, b_111,
           W1_0, b1_0, g_0, beta_0, W2_0, b2_0,
           W1_1, b1_1, g_1, beta_1, W2_1, b2_1):
    del inverse_edge1
    N, D = edge_attr0.shape
    E = edge_attr1.shape[0]

    src = edge_index1[0]
    dst = edge_index1[1]
    t0 = tri111[0]
    t1 = tri111[1]
    t2 = tri111[2]

    hb1, hr1 = _pre(edge_attr1, W_inner, b_inner, block=512)
    hb0, _ = _pre(edge_attr0, W_inner, b_inner, block=400)

    zeros = jnp.zeros((_CW, D), jnp.float32)

    g01, p = _sc_edges(hb0, hr1, src, dst, zeros)
    tri_agg = _sc_tri(hb1, t0, t1, t2, zeros)

    out1 = _out1(edge_attr1, g01, tri_agg, W_111, b_111,
                 W1_1, b1_1, g_1, beta_1, W2_1, b2_1, block=512)
    out0 = _out0(edge_attr0, p[:N], p[N:], W1_0, b1_0, g_0, beta_0,
                 W2_0, b2_0, block=400)
    return (out0, out1)
